# baseline (speedup 1.0000x reference)
import functools

import jax
import jax.numpy as jnp
from jax import lax
from jax.experimental import pallas as pl
from jax.experimental.pallas import tpu as pltpu
from jax.experimental.pallas import tpu_sc as plsc

_NUM_CORES = 2
_NUM_SUBCORES = 16
_LANES = 16
_NUM_WORKERS = _NUM_CORES * _NUM_SUBCORES

_B = 16384
_D_T, _D_C, _D_R = 16, 8, 8
_D_OUT = _D_T + _D_C + _D_R
_B_PER_W = _B // _NUM_WORKERS
_GROUPS = _B_PER_W // _LANES


def _body(tid_hbm, cid_hbm, rid_hbm, tw_hbm, cw_hbm, rw_hbm, out_hbm,
          tid_v, cid_v, rid_v, tw_v, cw_v, rw_v, out_v):
    wid = lax.axis_index("s") * _NUM_CORES + lax.axis_index("c")
    base = wid * _B_PER_W

    pltpu.sync_copy(tid_hbm.at[pl.ds(base, _B_PER_W)], tid_v)
    pltpu.sync_copy(cid_hbm.at[pl.ds(base, _B_PER_W)], cid_v)
    pltpu.sync_copy(rid_hbm.at[pl.ds(base, _B_PER_W)], rid_v)
    pltpu.sync_copy(tw_hbm, tw_v)
    pltpu.sync_copy(cw_hbm, cw_v)
    pltpu.sync_copy(rw_hbm, rw_v)

    lanes = lax.iota(jnp.int32, _LANES)

    def group(g, _):
        row = g * _LANES + lanes
        tv = tid_v[pl.ds(g * _LANES, _LANES)]
        cv = cid_v[pl.ds(g * _LANES, _LANES)]
        rv = rid_v[pl.ds(g * _LANES, _LANES)]
        for j in range(_D_T):
            col = jnp.full((_LANES,), j, jnp.int32)
            plsc.store_scatter(out_v, [row, col], plsc.load_gather(tw_v, [tv, col]))
        for j in range(_D_C):
            col = jnp.full((_LANES,), j, jnp.int32)
            x = plsc.load_gather(cw_v, [cv, col])
            plsc.store_scatter(out_v, [row, col + _D_T], x)
        for j in range(_D_R):
            col = jnp.full((_LANES,), j, jnp.int32)
            x = plsc.load_gather(rw_v, [rv, col])
            plsc.store_scatter(out_v, [row, col + _D_T + _D_C], x)
        return 0

    lax.fori_loop(0, _GROUPS, group, 0)

    pltpu.sync_copy(out_v, out_hbm.at[pl.ds(base, _B_PER_W)])


@jax.jit
def _run(tid, cid, rid, tw, cw, rw):
    mesh = plsc.VectorSubcoreMesh(core_axis_name="c", subcore_axis_name="s")
    return pl.kernel(
        _body,
        out_type=jax.ShapeDtypeStruct((_B, _D_OUT), jnp.float32),
        mesh=mesh,
        compiler_params=pltpu.CompilerParams(needs_layout_passes=False),
        scratch_types=[
            pltpu.VMEM((_B_PER_W,), jnp.int32),
            pltpu.VMEM((_B_PER_W,), jnp.int32),
            pltpu.VMEM((_B_PER_W,), jnp.int32),
            pltpu.VMEM((5, _D_T), jnp.float32),
            pltpu.VMEM((2, _D_C), jnp.float32),
            pltpu.VMEM((2, _D_R), jnp.float32),
            pltpu.VMEM((_B_PER_W, _D_OUT), jnp.float32),
        ],
    )(tid, cid, rid, tw, cw, rw)


def kernel(timepoint_ids, condition_ids, region_ids, timepoint_weight,
           condition_weight, region_weight):
    return _run(
        jnp.asarray(timepoint_ids, jnp.int32),
        jnp.asarray(condition_ids, jnp.int32),
        jnp.asarray(region_ids, jnp.int32),
        timepoint_weight,
        condition_weight,
        region_weight,
    )
